# Initial kernel scaffold; baseline (speedup 1.0000x reference)
import jax
import jax.numpy as jnp
from jax.experimental import pallas as pl


def kernel(src, tar):
    raise NotImplementedError("write your pallas kernel here")


# speedup vs baseline: 162.9753x; 162.9753x over previous
import functools

import jax
import jax.numpy as jnp
from jax import lax
from jax.experimental import pallas as pl
from jax.experimental.pallas import tpu as pltpu
from jax.experimental.pallas import tpu_sc as plsc

_LAMBDA_L2 = 1.0
_N = 128 * 1 * 192 * 84
_BATCH = 128
_NW = 32
_PER_W = _N // _NW
_CHUNK = 16128
_NCHUNK = _PER_W // _CHUNK
_VECS = _CHUNK // 16
_NBINS = 256
_HIST = 16 * _NBINS

_mesh = plsc.VectorSubcoreMesh(core_axis_name="c", subcore_axis_name="s")


@functools.partial(
    pl.kernel,
    mesh=_mesh,
    out_type=jax.ShapeDtypeStruct((_NW, 2, _HIST), jnp.float32),
    scratch_types=[
        pltpu.VMEM((_CHUNK,), jnp.float32),
        pltpu.VMEM((_CHUNK,), jnp.float32),
        pltpu.VMEM((_HIST,), jnp.float32),
        pltpu.VMEM((_HIST,), jnp.float32),
    ],
    compiler_params=pltpu.CompilerParams(needs_layout_passes=False),
)
def _sc_bin_pass(src_hbm, tar_hbm, out_hbm, sbuf, tbuf, hsum, hcnt):
    wid = lax.axis_index("s") * 2 + lax.axis_index("c")
    base = wid * _PER_W
    zeros = jnp.zeros((16,), jnp.float32)
    ones = jnp.ones((16,), jnp.float32)
    lane_off = lax.iota(jnp.int32, 16) * _NBINS

    def zero_body(i, carry):
        hsum[pl.ds(i * 16, 16)] = zeros
        hcnt[pl.ds(i * 16, 16)] = zeros
        return carry

    lax.fori_loop(0, _HIST // 16, zero_body, 0)

    for j in range(_NCHUNK):
        off = base + j * _CHUNK
        pltpu.sync_copy(src_hbm.at[pl.ds(off, _CHUNK)], sbuf)
        pltpu.sync_copy(tar_hbm.at[pl.ds(off, _CHUNK)], tbuf)

        def body(i, carry):
            t = tbuf[pl.ds(i * 16, 16)]
            s = sbuf[pl.ds(i * 16, 16)]
            d = (s - t) * (s - t)
            b = t.astype(jnp.int32) + lane_off
            plsc.addupdate_scatter(hsum, [b], d)
            plsc.addupdate_scatter(hcnt, [b], ones)
            return carry

        lax.fori_loop(0, _VECS, body, 0)

    pltpu.sync_copy(hsum, out_hbm.at[wid, 0])
    pltpu.sync_copy(hcnt, out_hbm.at[wid, 1])


def _tc_epilogue(s_ref, c_ref, o_ref):
    sums = jnp.sum(s_ref[...], axis=0, keepdims=True)
    counts = jnp.sum(c_ref[...], axis=0, keepdims=True)
    p_y = counts * (1.0 / (float(_N) * float(_BATCH)))
    w = 1.0 / (p_y + 1e-12)
    w = w / jnp.sum(w)
    loss = jnp.sum(w * sums) * _LAMBDA_L2
    o_ref[...] = jnp.reshape(loss, (1, 1))


def kernel(src, tar):
    s = src.reshape(-1)
    t = tar.reshape(-1)
    parts = _sc_bin_pass(s, t)
    s_in = parts[:, 0, :].reshape(_NW * 16, _NBINS)
    c_in = parts[:, 1, :].reshape(_NW * 16, _NBINS)
    loss = pl.pallas_call(
        _tc_epilogue,
        out_shape=jax.ShapeDtypeStruct((1, 1), jnp.float32),
    )(s_in, c_in)
    return loss[0, 0]


# speedup vs baseline: 204.8580x; 1.2570x over previous
import functools

import jax
import jax.numpy as jnp
from jax import lax
from jax.experimental import pallas as pl
from jax.experimental.pallas import tpu as pltpu
from jax.experimental.pallas import tpu_sc as plsc

_LAMBDA_L2 = 1.0
_N = 128 * 1 * 192 * 84
_BATCH = 128
_NW = 32
_PER_W = _N // _NW
_CHUNK = 16128
_NCHUNK = _PER_W // _CHUNK
_VECS = _CHUNK // 16
_NBINS = 256
_HIST = 16 * _NBINS

_mesh = plsc.VectorSubcoreMesh(core_axis_name="c", subcore_axis_name="s")


@functools.partial(
    pl.kernel,
    mesh=_mesh,
    out_type=jax.ShapeDtypeStruct((_NW, 2, _HIST), jnp.float32),
    scratch_types=[
        pltpu.VMEM((_CHUNK,), jnp.float32),
        pltpu.VMEM((_CHUNK,), jnp.float32),
        pltpu.VMEM((_HIST,), jnp.float32),
        pltpu.VMEM((_HIST,), jnp.float32),
    ],
    compiler_params=pltpu.CompilerParams(needs_layout_passes=False),
)
def _sc_bin_pass(src_hbm, tar_hbm, out_hbm, sbuf, tbuf, hsum, hcnt):
    wid = lax.axis_index("s") * 2 + lax.axis_index("c")
    base = wid * _PER_W
    zeros = jnp.zeros((16,), jnp.float32)
    ones = jnp.ones((16,), jnp.float32)
    lane_off = lax.iota(jnp.int32, 16) * _NBINS

    @plsc.parallel_loop(0, _HIST // 16, unroll=4)
    def zero_body(i):
        hsum[pl.ds(i * 16, 16)] = zeros
        hcnt[pl.ds(i * 16, 16)] = zeros

    for j in range(_NCHUNK):
        off = base + j * _CHUNK
        pltpu.sync_copy(src_hbm.at[pl.ds(off, _CHUNK)], sbuf)
        pltpu.sync_copy(tar_hbm.at[pl.ds(off, _CHUNK)], tbuf)

        @plsc.parallel_loop(0, _VECS, unroll=8)
        def body(i):
            t = tbuf[pl.ds(i * 16, 16)]
            s = sbuf[pl.ds(i * 16, 16)]
            d = (s - t) * (s - t)
            b = t.astype(jnp.int32) + lane_off
            plsc.addupdate_scatter(hsum, [b], d)
            plsc.addupdate_scatter(hcnt, [b], ones)

    pltpu.sync_copy(hsum, out_hbm.at[wid, 0])
    pltpu.sync_copy(hcnt, out_hbm.at[wid, 1])


def _tc_epilogue(s_ref, c_ref, o_ref):
    sums = jnp.sum(s_ref[...], axis=0, keepdims=True)
    counts = jnp.sum(c_ref[...], axis=0, keepdims=True)
    p_y = counts * (1.0 / (float(_N) * float(_BATCH)))
    w = 1.0 / (p_y + 1e-12)
    w = w / jnp.sum(w)
    loss = jnp.sum(w * sums) * _LAMBDA_L2
    o_ref[...] = jnp.reshape(loss, (1, 1))


def kernel(src, tar):
    s = src.reshape(-1)
    t = tar.reshape(-1)
    parts = _sc_bin_pass(s, t)
    s_in = parts[:, 0, :].reshape(_NW * 16, _NBINS)
    c_in = parts[:, 1, :].reshape(_NW * 16, _NBINS)
    loss = pl.pallas_call(
        _tc_epilogue,
        out_shape=jax.ShapeDtypeStruct((1, 1), jnp.float32),
    )(s_in, c_in)
    return loss[0, 0]
